# Initial kernel scaffold; baseline (speedup 1.0000x reference)
import jax
import jax.numpy as jnp
from jax.experimental import pallas as pl


def kernel(x, expert_ids, smooth_scales, expert_scales, x_active_mask, gmm1_weight, gmm2_weight):
    raise NotImplementedError("write your pallas kernel here")


# speedup vs baseline: 1.3614x; 1.3614x over previous
import jax
import jax.numpy as jnp
from jax.experimental import pallas as pl

B = 128
H = 2048
I = 1024
K = 8
LOCAL = 8


def _mlp1_body(x_ref, w1_ref, act_ref):
    w1 = w1_ref[0].reshape(H, 2 * I)
    h1 = jnp.dot(x_ref[...], w1, preferred_element_type=jnp.float32)
    gate = h1[:, :I]
    up = h1[:, I:]
    act_ref[0] = gate * jax.nn.sigmoid(gate) * up


def _mlp2_body(act_ref, w2_ref, eid_ref, sc_ref, out_ref):
    e = pl.program_id(0)
    eid = eid_ref[...]
    sc = sc_ref[...]
    w = jnp.sum(jnp.where(eid == e, sc, 0.0), axis=1)
    a = act_ref[0] * w[:, None]
    contrib = jnp.dot(a, w2_ref[0], preferred_element_type=jnp.float32)

    @pl.when(e == 0)
    def _():
        out_ref[...] = jnp.zeros_like(out_ref)

    out_ref[...] += contrib


def kernel(x, expert_ids, smooth_scales, expert_scales, x_active_mask,
           gmm1_weight, gmm2_weight):
    del smooth_scales
    eids = expert_ids.astype(jnp.int32)
    sc = expert_scales * x_active_mask[:, None].astype(jnp.float32)
    w1 = gmm1_weight.reshape(LOCAL, H, 2, I)

    act = pl.pallas_call(
        _mlp1_body,
        grid=(LOCAL,),
        in_specs=[
            pl.BlockSpec((B, H), lambda e: (0, 0)),
            pl.BlockSpec((1, H, 2, I), lambda e: (e, 0, 0, 0)),
        ],
        out_specs=pl.BlockSpec((1, B, I), lambda e: (e, 0, 0)),
        out_shape=jax.ShapeDtypeStruct((LOCAL, B, I), jnp.float32),
    )(x, w1)

    out = pl.pallas_call(
        _mlp2_body,
        grid=(LOCAL,),
        in_specs=[
            pl.BlockSpec((1, B, I), lambda e: (e, 0, 0)),
            pl.BlockSpec((1, I, H), lambda e: (e, 0, 0)),
            pl.BlockSpec((B, K), lambda e: (0, 0)),
            pl.BlockSpec((B, K), lambda e: (0, 0)),
        ],
        out_specs=pl.BlockSpec((B, H), lambda e: (0, 0)),
        out_shape=jax.ShapeDtypeStruct((B, H), jnp.float32),
    )(act, gmm2_weight, eids, sc)
    return out
